# baseline (speedup 1.0000x reference)
import functools

import jax
import jax.numpy as jnp
from jax import lax
from jax.experimental import pallas as pl
from jax.experimental.pallas import tpu as pltpu
from jax.experimental.pallas import tpu_sc as plsc

VOCAB = 1000000
EMBED_DIM = 32
BATCH = 4096
HIST = 200

_info = plsc.get_sparse_core_info()
NC, NS, NL = _info.num_cores, _info.num_subcores, _info.num_lanes
NW = NC * NS
BTILE = BATCH // NW
ET = EMBED_DIM // 8
NP = HIST // 2

_mesh = plsc.VectorSubcoreMesh(core_axis_name="c", subcore_axis_name="s")


@functools.partial(
    pl.kernel,
    mesh=_mesh,
    out_type=jax.ShapeDtypeStruct((HIST, ET, NW, 8, BTILE), jnp.float32),
    scratch_types=[
        pltpu.VMEM((HIST, BTILE), jnp.int32),
        pltpu.VMEM((BTILE, EMBED_DIM), jnp.float32),
        pltpu.VMEM((BTILE, EMBED_DIM), jnp.float32),
        pltpu.VMEM((ET, 8, BTILE), jnp.float32),
        pltpu.VMEM((ET, 8, BTILE), jnp.float32),
        pltpu.SemaphoreType.DMA,
        pltpu.SemaphoreType.DMA,
        pltpu.SemaphoreType.DMA,
        pltpu.SemaphoreType.DMA,
    ],
    compiler_params=pltpu.CompilerParams(
        use_tc_tiling_on_sc=False, needs_layout_passes=False
    ),
)
def _embed(idx_hbm, table_hbm, out_hbm, idx_v, g_a, g_b, t_a, t_b,
           gsem_a, gsem_b, osem_a, osem_b):
    wid = lax.axis_index("s") * NC + lax.axis_index("c")
    pltpu.sync_copy(idx_hbm.at[wid], idx_v)

    iota = lax.iota(jnp.int32, NL)

    def fire_gather(h, gbuf, sem):
        pltpu.async_copy(table_hbm.at[idx_v.at[h]], gbuf, sem)

    def drain_gather(gbuf, sem):
        pltpu.make_async_copy(table_hbm.at[pl.ds(0, BTILE)], gbuf, sem).wait()

    def transpose(gbuf, tbuf):
        for te in range(ET):
            for ein in range(8):
                e_idx = jnp.full((NL,), te * 8 + ein, jnp.int32)
                for t0 in range(BTILE // NL):
                    t_idx = iota + (t0 * NL)
                    vec = plsc.load_gather(gbuf, [t_idx, e_idx])
                    tbuf[te, ein, pl.ds(t0 * NL, NL)] = vec

    def fire_out(h, tbuf, sem):
        for te in range(ET):
            pltpu.async_copy(tbuf.at[te], out_hbm.at[h, te, wid], sem)

    def drain_out(tbuf, sem):
        for te in range(ET):
            pltpu.make_async_copy(tbuf.at[te], out_hbm.at[0, te, wid],
                                  sem).wait()

    fire_gather(0, g_a, gsem_a)
    fire_gather(1, g_b, gsem_b)

    def step(p, carry):
        h0 = 2 * p

        def half(h, gbuf, tbuf, gsem, osem):
            drain_gather(gbuf, gsem)

            @pl.when(p > 0)
            def _():
                drain_out(tbuf, osem)

            transpose(gbuf, tbuf)

            @pl.when(p < NP - 1)
            def _():
                fire_gather(h + 2, gbuf, gsem)

            fire_out(h, tbuf, osem)

        half(h0, g_a, t_a, gsem_a, osem_a)
        half(h0 + 1, g_b, t_b, gsem_b, osem_b)
        return carry

    lax.fori_loop(0, NP, step, 0)
    drain_out(t_a, osem_a)
    drain_out(t_b, osem_b)


def kernel(weight, tensor0):
    idx3 = tensor0.T.reshape(HIST, NW, BTILE).transpose(1, 0, 2)
    out6 = _embed(idx3, weight)
    return out6.transpose(2, 4, 0, 1, 3).reshape(BATCH, HIST, EMBED_DIM)


# speedup vs baseline: 1.1906x; 1.1906x over previous
import functools

import jax
import jax.numpy as jnp
from jax import lax
from jax.experimental import pallas as pl
from jax.experimental.pallas import tpu as pltpu
from jax.experimental.pallas import tpu_sc as plsc

VOCAB = 1000000
EMBED_DIM = 32
BATCH = 4096
HIST = 200

_info = plsc.get_sparse_core_info()
NC, NS, NL = _info.num_cores, _info.num_subcores, _info.num_lanes
NW = NC * NS
BTILE = BATCH // NW
ET = EMBED_DIM // 8
NP = HIST // 2

_mesh = plsc.VectorSubcoreMesh(core_axis_name="c", subcore_axis_name="s")


@functools.partial(
    pl.kernel,
    mesh=_mesh,
    out_type=jax.ShapeDtypeStruct((HIST, ET, NW, 8, BTILE), jnp.float32),
    scratch_types=[
        pltpu.VMEM((HIST, BTILE), jnp.int32),
        pltpu.VMEM((BTILE, EMBED_DIM), jnp.float32),
        pltpu.VMEM((BTILE, EMBED_DIM), jnp.float32),
        pltpu.VMEM((ET, 8, BTILE), jnp.float32),
        pltpu.VMEM((ET, 8, BTILE), jnp.float32),
        pltpu.SemaphoreType.DMA,
        pltpu.SemaphoreType.DMA,
        pltpu.SemaphoreType.DMA,
        pltpu.SemaphoreType.DMA,
    ],
    compiler_params=pltpu.CompilerParams(
        use_tc_tiling_on_sc=False, needs_layout_passes=False
    ),
)
def _embed(idx_hbm, table_hbm, out_hbm, idx_v, g_a, g_b, t_a, t_b,
           gsem_a, gsem_b, osem_a, osem_b):
    wid = lax.axis_index("s") * NC + lax.axis_index("c")
    pltpu.sync_copy(idx_hbm.at[wid], idx_v)

    iota = lax.iota(jnp.int32, NL)

    def fire_gather(h, gbuf, sem):
        pltpu.async_copy(table_hbm.at[idx_v.at[h]], gbuf, sem)

    def drain_gather(gbuf, sem):
        pltpu.make_async_copy(table_hbm.at[pl.ds(0, BTILE)], gbuf, sem).wait()

    def transpose(gbuf, tbuf):
        for te in range(ET):
            for ein in range(8):
                e_idx = jnp.full((NL,), te * 8 + ein, jnp.int32)
                vecs = [
                    plsc.load_gather(gbuf, [iota + (t0 * NL), e_idx])
                    for t0 in range(BTILE // NL)
                ]
                for t0, vec in enumerate(vecs):
                    tbuf[te, ein, pl.ds(t0 * NL, NL)] = vec

    def fire_out(h, tbuf, sem):
        for te in range(ET):
            pltpu.async_copy(tbuf.at[te], out_hbm.at[h, te, wid], sem)

    def drain_out(tbuf, sem):
        for te in range(ET):
            pltpu.make_async_copy(tbuf.at[te], out_hbm.at[0, te, wid],
                                  sem).wait()

    fire_gather(0, g_a, gsem_a)
    fire_gather(1, g_b, gsem_b)

    def step(p, carry):
        h0 = 2 * p

        def half(h, gbuf, tbuf, gsem, osem):
            drain_gather(gbuf, gsem)

            @pl.when(p > 0)
            def _():
                drain_out(tbuf, osem)

            transpose(gbuf, tbuf)

            @pl.when(p < NP - 1)
            def _():
                fire_gather(h + 2, gbuf, gsem)

            fire_out(h, tbuf, osem)

        half(h0, g_a, t_a, gsem_a, osem_a)
        half(h0 + 1, g_b, t_b, gsem_b, osem_b)
        return carry

    lax.fori_loop(0, NP, step, 0)
    drain_out(t_a, osem_a)
    drain_out(t_b, osem_b)


def kernel(weight, tensor0):
    idx3 = tensor0.T.reshape(HIST, NW, BTILE).transpose(1, 0, 2)
    out6 = _embed(idx3, weight)
    return out6.transpose(2, 4, 0, 1, 3).reshape(BATCH, HIST, EMBED_DIM)


# speedup vs baseline: 1.6982x; 1.4264x over previous
import functools

import jax
import jax.numpy as jnp
from jax import lax
from jax.experimental import pallas as pl
from jax.experimental.pallas import tpu as pltpu
from jax.experimental.pallas import tpu_sc as plsc

VOCAB = 1000000
EMBED_DIM = 32
BATCH = 4096
HIST = 200

_info = plsc.get_sparse_core_info()
NC, NS, NL = _info.num_cores, _info.num_subcores, _info.num_lanes
NW = NC * NS
BTILE = BATCH // NW
ET = EMBED_DIM // 8
NP = HIST // 2

_mesh = plsc.VectorSubcoreMesh(core_axis_name="c", subcore_axis_name="s")


@functools.partial(
    pl.kernel,
    mesh=_mesh,
    out_type=jax.ShapeDtypeStruct((HIST, ET, NW, 8, BTILE), jnp.float32),
    scratch_types=[
        pltpu.VMEM((HIST, BTILE), jnp.int32),
        pltpu.VMEM((BTILE, EMBED_DIM), jnp.float32),
        pltpu.VMEM((BTILE, EMBED_DIM), jnp.float32),
        pltpu.VMEM((EMBED_DIM, BTILE + 1), jnp.float32),
        pltpu.VMEM((EMBED_DIM, BTILE + 1), jnp.float32),
        pltpu.SemaphoreType.DMA,
        pltpu.SemaphoreType.DMA,
        pltpu.SemaphoreType.DMA,
        pltpu.SemaphoreType.DMA,
    ],
    compiler_params=pltpu.CompilerParams(
        use_tc_tiling_on_sc=False, needs_layout_passes=False
    ),
)
def _embed(idx_hbm, table_hbm, out_hbm, idx_v, g_a, g_b, t_a, t_b,
           gsem_a, gsem_b, osem_a, osem_b):
    wid = lax.axis_index("s") * NC + lax.axis_index("c")
    pltpu.sync_copy(idx_hbm.at[wid], idx_v)

    iota = lax.iota(jnp.int32, NL)

    def fire_gather(h, gbuf, sem):
        pltpu.async_copy(table_hbm.at[idx_v.at[h]], gbuf, sem)

    def drain_gather(gbuf, sem):
        pltpu.make_async_copy(table_hbm.at[pl.ds(0, BTILE)], gbuf, sem).wait()

    def transpose(gbuf, tbuf):
        for tg in range(0, BTILE, 8):
            vecs = [
                (t, half, gbuf[t, pl.ds(half * NL, NL)])
                for t in range(tg, tg + 8)
                for half in range(2)
            ]
            for t, half, vec in vecs:
                e_idx = iota + (half * NL)
                t_idx = jnp.full((NL,), t, jnp.int32)
                plsc.store_scatter(tbuf, [e_idx, t_idx], vec)

    def fire_out(h, tbuf, sem):
        for te in range(ET):
            pltpu.async_copy(
                tbuf.at[pl.ds(te * 8, 8), pl.ds(0, BTILE)],
                out_hbm.at[h, te, wid],
                sem,
            )

    def drain_out(tbuf, sem):
        for te in range(ET):
            pltpu.make_async_copy(
                tbuf.at[pl.ds(te * 8, 8), pl.ds(0, BTILE)],
                out_hbm.at[0, te, wid],
                sem,
            ).wait()

    fire_gather(0, g_a, gsem_a)
    fire_gather(1, g_b, gsem_b)

    def step(p, carry):
        h0 = 2 * p

        def half(h, gbuf, tbuf, gsem, osem):
            drain_gather(gbuf, gsem)

            @pl.when(p > 0)
            def _():
                drain_out(tbuf, osem)

            transpose(gbuf, tbuf)

            @pl.when(p < NP - 1)
            def _():
                fire_gather(h + 2, gbuf, gsem)

            fire_out(h, tbuf, osem)

        half(h0, g_a, t_a, gsem_a, osem_a)
        half(h0 + 1, g_b, t_b, gsem_b, osem_b)
        return carry

    lax.fori_loop(0, NP, step, 0)
    drain_out(t_a, osem_a)
    drain_out(t_b, osem_b)


def kernel(weight, tensor0):
    idx3 = tensor0.T.reshape(HIST, NW, BTILE).transpose(1, 0, 2)
    out6 = _embed(idx3, weight)
    return out6.transpose(2, 4, 0, 1, 3).reshape(BATCH, HIST, EMBED_DIM)


# speedup vs baseline: 1.8423x; 1.0849x over previous
import functools

import jax
import jax.numpy as jnp
import numpy as np
from jax import lax
from jax.experimental import pallas as pl
from jax.experimental.pallas import tpu as pltpu
from jax.experimental.pallas import tpu_sc as plsc

VOCAB = 1000000
EMBED_DIM = 32
BATCH = 4096
HIST = 200

_info = plsc.get_sparse_core_info()
NC, NS, NL = _info.num_cores, _info.num_subcores, _info.num_lanes
NW = NC * NS
BTILE = BATCH // NW
ET = EMBED_DIM // 8
NP = HIST // 2

_mesh = plsc.VectorSubcoreMesh(core_axis_name="c", subcore_axis_name="s")


@functools.partial(
    pl.kernel,
    mesh=_mesh,
    out_type=jax.ShapeDtypeStruct((HIST, ET, NW, 8, BTILE), jnp.float32),
    scratch_types=[
        pltpu.VMEM((HIST, BTILE), jnp.int32),
        pltpu.VMEM((BTILE, NL), jnp.int32),
        pltpu.VMEM((BTILE, EMBED_DIM), jnp.float32),
        pltpu.VMEM((BTILE, EMBED_DIM), jnp.float32),
        pltpu.VMEM((EMBED_DIM, BTILE + 1), jnp.float32),
        pltpu.VMEM((EMBED_DIM, BTILE + 1), jnp.float32),
        pltpu.SemaphoreType.DMA,
        pltpu.SemaphoreType.DMA,
        pltpu.SemaphoreType.DMA,
        pltpu.SemaphoreType.DMA,
    ],
    compiler_params=pltpu.CompilerParams(
        use_tc_tiling_on_sc=False, needs_layout_passes=False
    ),
)
def _embed(idx_hbm, tconst_hbm, table_hbm, out_hbm, idx_v, tc_v, g_a, g_b,
           t_a, t_b, gsem_a, gsem_b, osem_a, osem_b):
    wid = lax.axis_index("s") * NC + lax.axis_index("c")
    pltpu.sync_copy(idx_hbm.at[wid], idx_v)
    pltpu.sync_copy(tconst_hbm, tc_v)

    iota = lax.iota(jnp.int32, NL)

    def fire_gather(h, gbuf, sem):
        pltpu.async_copy(table_hbm.at[idx_v.at[h]], gbuf, sem)

    def drain_gather(gbuf, sem):
        pltpu.make_async_copy(table_hbm.at[pl.ds(0, BTILE)], gbuf, sem).wait()

    def transpose(gbuf, tbuf):
        for tg in range(0, BTILE, 8):
            vecs = [
                (t, half, gbuf[t, pl.ds(half * NL, NL)], tc_v[t])
                for t in range(tg, tg + 8)
                for half in range(2)
            ]
            for t, half, vec, t_idx in vecs:
                e_idx = iota + (half * NL)
                plsc.store_scatter(tbuf, [e_idx, t_idx], vec)

    def fire_out(h, tbuf, sem):
        for te in range(ET):
            pltpu.async_copy(
                tbuf.at[pl.ds(te * 8, 8), pl.ds(0, BTILE)],
                out_hbm.at[h, te, wid],
                sem,
            )

    def drain_out(tbuf, sem):
        for te in range(ET):
            pltpu.make_async_copy(
                tbuf.at[pl.ds(te * 8, 8), pl.ds(0, BTILE)],
                out_hbm.at[0, te, wid],
                sem,
            ).wait()

    fire_gather(0, g_a, gsem_a)
    fire_gather(1, g_b, gsem_b)

    def step(p, carry):
        h0 = 2 * p

        def half(h, gbuf, tbuf, gsem, osem):
            drain_gather(gbuf, gsem)

            @pl.when(p > 0)
            def _():
                drain_out(tbuf, osem)

            transpose(gbuf, tbuf)

            @pl.when(p < NP - 1)
            def _():
                fire_gather(h + 2, gbuf, gsem)

            fire_out(h, tbuf, osem)

        half(h0, g_a, t_a, gsem_a, osem_a)
        half(h0 + 1, g_b, t_b, gsem_b, osem_b)
        return carry

    lax.fori_loop(0, NP, step, 0)
    drain_out(t_a, osem_a)
    drain_out(t_b, osem_b)


def kernel(weight, tensor0):
    idx3 = tensor0.T.reshape(HIST, NW, BTILE).transpose(1, 0, 2)
    tconst = jnp.broadcast_to(
        jnp.arange(BTILE, dtype=jnp.int32)[:, None], (BTILE, NL)
    )
    out6 = _embed(idx3, tconst, weight)
    return out6.transpose(2, 4, 0, 1, 3).reshape(BATCH, HIST, EMBED_DIM)


# speedup vs baseline: 1.8426x; 1.0001x over previous
import functools

import jax
import jax.numpy as jnp
from jax import lax
from jax.experimental import pallas as pl
from jax.experimental.pallas import tpu as pltpu
from jax.experimental.pallas import tpu_sc as plsc

VOCAB = 1000000
EMBED_DIM = 32
BATCH = 4096
HIST = 200

_info = plsc.get_sparse_core_info()
NC, NS, NL = _info.num_cores, _info.num_subcores, _info.num_lanes
NW = NC * NS
BTILE = BATCH // NW
ET = EMBED_DIM // 8
NP = HIST // 2

_mesh = plsc.VectorSubcoreMesh(core_axis_name="c", subcore_axis_name="s")


@functools.partial(
    pl.kernel,
    mesh=_mesh,
    out_type=jax.ShapeDtypeStruct((HIST, ET, NW, 8, BTILE), jnp.float32),
    scratch_types=[
        pltpu.VMEM((HIST, BTILE), jnp.int32),
        pltpu.VMEM((BTILE, NL), jnp.int32),
        pltpu.VMEM((BTILE, EMBED_DIM), jnp.float32),
        pltpu.VMEM((BTILE, EMBED_DIM), jnp.float32),
        pltpu.VMEM((EMBED_DIM, BTILE + 1), jnp.float32),
        pltpu.VMEM((EMBED_DIM, BTILE + 1), jnp.float32),
        pltpu.SemaphoreType.DMA,
        pltpu.SemaphoreType.DMA,
        pltpu.SemaphoreType.DMA,
        pltpu.SemaphoreType.DMA,
    ],
    compiler_params=pltpu.CompilerParams(
        use_tc_tiling_on_sc=False, needs_layout_passes=False
    ),
)
def _embed(idx_hbm, tconst_hbm, table_hbm, out_hbm, idx_v, tc_v, g_a, g_b,
           t_a, t_b, gsem_a, gsem_b, osem_a, osem_b):
    wid = lax.axis_index("s") * NC + lax.axis_index("c")
    pltpu.sync_copy(idx_hbm.at[wid], idx_v)
    pltpu.sync_copy(tconst_hbm, tc_v)

    iota = lax.iota(jnp.int32, NL)

    def fire_gather(h, gbuf, sem):
        pltpu.async_copy(table_hbm.at[idx_v.at[h]], gbuf, sem)

    def drain_gather(gbuf, sem):
        pltpu.make_async_copy(table_hbm.at[pl.ds(0, BTILE)], gbuf, sem).wait()

    def transpose(gbuf, tbuf):
        for tg in range(0, BTILE, 8):
            vecs = [
                (tc_v[t], gbuf[t, pl.ds(0, NL)], gbuf[t, pl.ds(NL, NL)])
                for t in range(tg, tg + 8)
            ]
            for t_idx, lo, hi in vecs:
                plsc.store_scatter(tbuf, [iota, t_idx], lo)
                plsc.store_scatter(tbuf, [iota + NL, t_idx], hi)

    def fire_out(h, tbuf, sem):
        for te in range(ET):
            pltpu.async_copy(
                tbuf.at[pl.ds(te * 8, 8), pl.ds(0, BTILE)],
                out_hbm.at[h, te, wid],
                sem,
            )

    def drain_out(tbuf, sem):
        for te in range(ET):
            pltpu.make_async_copy(
                tbuf.at[pl.ds(te * 8, 8), pl.ds(0, BTILE)],
                out_hbm.at[0, te, wid],
                sem,
            ).wait()

    fire_gather(0, g_a, gsem_a)
    fire_gather(1, g_b, gsem_b)

    def step(p, carry):
        h0 = 2 * p

        def half(h, gbuf, tbuf, gsem, osem):
            drain_gather(gbuf, gsem)

            @pl.when(p > 0)
            def _():
                drain_out(tbuf, osem)

            transpose(gbuf, tbuf)

            @pl.when(p < NP - 1)
            def _():
                fire_gather(h + 2, gbuf, gsem)

            fire_out(h, tbuf, osem)

        half(h0, g_a, t_a, gsem_a, osem_a)
        half(h0 + 1, g_b, t_b, gsem_b, osem_b)
        return carry

    lax.fori_loop(0, NP, step, 0)
    drain_out(t_a, osem_a)
    drain_out(t_b, osem_b)


def kernel(weight, tensor0):
    idx3 = tensor0.T.reshape(HIST, NW, BTILE).transpose(1, 0, 2)
    tconst = jnp.broadcast_to(
        jnp.arange(BTILE, dtype=jnp.int32)[:, None], (BTILE, NL)
    )
    out6 = _embed(idx3, tconst, weight)
    return out6.transpose(2, 4, 0, 1, 3).reshape(BATCH, HIST, EMBED_DIM)
